# baseline (speedup 1.0000x reference)
import functools

import jax
import jax.numpy as jnp
from jax import lax
from jax.experimental import pallas as pl
from jax.experimental.pallas import tpu as pltpu
from jax.experimental.pallas import tpu_sc as plsc

_CHUNK = 128
_NBUF = 5


def _make_gather(num_rows, d, n_out, n_chunks, nw):
  assert n_chunks % nw == 0
  chunks_per_w = n_chunks // nw
  n_bblk = n_out // _CHUNK
  mesh = plsc.VectorSubcoreMesh(core_axis_name="c", subcore_axis_name="s")
  nc = 2

  @functools.partial(
      pl.kernel,
      mesh=mesh,
      out_type=jax.ShapeDtypeStruct((n_chunks * d, _CHUNK), jnp.float32),
      scratch_types=[
          pltpu.VMEM((chunks_per_w, _CHUNK), jnp.int32),
          pltpu.VMEM((d, _CHUNK), jnp.float32),
      ] + [pltpu.VMEM((_CHUNK, d), jnp.float32) for _ in range(_NBUF)] + [
          pltpu.SemaphoreType.DMA for _ in range(_NBUF)
      ],
      compiler_params=pltpu.CompilerParams(
          use_tc_tiling_on_sc=False, needs_layout_passes=False),
  )
  def gather_kernel(table_hbm, idx_hbm, out_hbm, idx_v, stage, *bufs_sems):
    wid = lax.axis_index("s") * nc + lax.axis_index("c")
    pltpu.sync_copy(idx_hbm.at[wid], idx_v)

    bufs = bufs_sems[:_NBUF]
    sems = bufs_sems[_NBUF:]
    depth = _NBUF - 1
    lanes = jax.lax.iota(jnp.int32, 16)

    def issue(j, p):
      pltpu.async_copy(table_hbm.at[idx_v.at[j]], bufs[p], sems[p])

    def drain(j, p):
      pltpu.make_async_copy(table_hbm.at[idx_v.at[j]], bufs[p], sems[p]).wait()
      g = bufs[p]
      def tr(m, carry):
        del carry
        rows = m * 16 + lanes
        for c in range(d):
          vals = plsc.load_gather(g, [rows, jnp.full((16,), c, jnp.int32)])
          stage[c, pl.ds(m * 16, 16)] = vals
        return 0

      lax.fori_loop(0, _CHUNK // 16, tr, 0)

      k = wid * chunks_per_w + j
      jj = k // n_bblk
      bk = k % n_bblk
      for cg in range(d // 8):
        r0 = pl.multiple_of(jj * (d * n_bblk) + cg * (8 * n_bblk) + bk * 8, 8)
        pltpu.sync_copy(stage.at[pl.ds(cg * 8, 8)], out_hbm.at[pl.ds(r0, 8)])

    for p in range(depth):
      issue(p, p)

    def body(g_, carry):
      del carry
      j = g_ * _NBUF
      for p in range(_NBUF):
        drain(j + p, p)

        @pl.when(j + p + depth < chunks_per_w)
        def _():
          issue(j + p + depth, (p + depth) % _NBUF)

      return 0

    assert chunks_per_w % _NBUF == 0
    lax.fori_loop(0, chunks_per_w // _NBUF, body, 0)

  return gather_kernel


def kernel(input_tensor, dim, indices):
  data = input_tensor
  dim_size = data.shape[0]
  d = data.shape[1]
  n_out, n_j = indices.shape
  flat_idx = indices.T.reshape(-1).astype(jnp.int32) + jnp.asarray(
      dim, dtype=jnp.int32)
  flat_idx = jnp.where(flat_idx < 0, flat_idx + dim_size, flat_idx)
  b = flat_idx.shape[0]
  n_chunks = b // _CHUNK
  n_bblk = n_out // _CHUNK

  info = plsc.get_sparse_core_info()
  nw = info.num_cores * info.num_subcores
  idx3 = flat_idx.reshape(nw, n_chunks // nw, _CHUNK)
  out2 = _make_gather(dim_size, d, n_out, n_chunks, nw)(
      data.astype(jnp.float32), idx3)
  out = (
      out2.reshape(n_j, d // 8, n_bblk, 8, _CHUNK)
      .transpose(2, 4, 0, 1, 3)
      .reshape(n_out, n_j, d)
  )
  return out.astype(data.dtype)


# speedup vs baseline: 1.0226x; 1.0226x over previous
import functools

import jax
import jax.numpy as jnp
from jax import lax
from jax.experimental import pallas as pl
from jax.experimental.pallas import tpu as pltpu
from jax.experimental.pallas import tpu_sc as plsc

_CHUNK = 128
_NBUF = 5


def _make_gather(num_rows, d, n_out, n_chunks, nw):
  assert n_chunks % nw == 0
  chunks_per_w = n_chunks // nw
  n_bblk = n_out // _CHUNK
  mesh = plsc.VectorSubcoreMesh(core_axis_name="c", subcore_axis_name="s")
  nc = 2

  @functools.partial(
      pl.kernel,
      mesh=mesh,
      out_type=jax.ShapeDtypeStruct(
          (n_chunks // n_bblk * (d // 8), n_bblk, 8, _CHUNK), jnp.float32),
      scratch_types=[
          pltpu.VMEM((chunks_per_w, _CHUNK), jnp.int32),
          pltpu.VMEM((d // 8, 8, _CHUNK), jnp.float32),
      ] + [pltpu.VMEM((_CHUNK, d), jnp.float32) for _ in range(_NBUF)] + [
          pltpu.SemaphoreType.DMA for _ in range(_NBUF)
      ],
      compiler_params=pltpu.CompilerParams(
          use_tc_tiling_on_sc=False, needs_layout_passes=False),
  )
  def gather_kernel(table_hbm, idx_hbm, out_hbm, idx_v, stage, *bufs_sems):
    wid = lax.axis_index("s") * nc + lax.axis_index("c")
    pltpu.sync_copy(idx_hbm.at[wid], idx_v)

    bufs = bufs_sems[:_NBUF]
    sems = bufs_sems[_NBUF:]
    depth = _NBUF - 1
    lanes = jax.lax.iota(jnp.int32, 16)

    def issue(j, p):
      pltpu.async_copy(table_hbm.at[idx_v.at[j]], bufs[p], sems[p])

    def drain(j, p):
      pltpu.make_async_copy(table_hbm.at[idx_v.at[j]], bufs[p], sems[p]).wait()
      g = bufs[p]
      def tr(m, carry):
        del carry
        rows = m * 16 + lanes
        for cg in range(d // 8):
          for c8 in range(8):
            vals = plsc.load_gather(
                g, [rows, jnp.full((16,), cg * 8 + c8, jnp.int32)])
            stage[cg, c8, pl.ds(m * 16, 16)] = vals
        return 0

      lax.fori_loop(0, _CHUNK // 16, tr, 0)

      k = wid * chunks_per_w + j
      jj = k // n_bblk
      bk = k % n_bblk
      pltpu.sync_copy(stage, out_hbm.at[pl.ds(jj * (d // 8), d // 8), bk])

    for p in range(depth):
      issue(p, p)

    def body(g_, carry):
      del carry
      j = g_ * _NBUF
      for p in range(_NBUF):
        drain(j + p, p)

        @pl.when(j + p + depth < chunks_per_w)
        def _():
          issue(j + p + depth, (p + depth) % _NBUF)

      return 0

    assert chunks_per_w % _NBUF == 0
    lax.fori_loop(0, chunks_per_w // _NBUF, body, 0)

  return gather_kernel


def kernel(input_tensor, dim, indices):
  data = input_tensor
  dim_size = data.shape[0]
  d = data.shape[1]
  n_out, n_j = indices.shape
  flat_idx = indices.T.reshape(-1).astype(jnp.int32) + jnp.asarray(
      dim, dtype=jnp.int32)
  flat_idx = jnp.where(flat_idx < 0, flat_idx + dim_size, flat_idx)
  b = flat_idx.shape[0]
  n_chunks = b // _CHUNK
  n_bblk = n_out // _CHUNK

  info = plsc.get_sparse_core_info()
  nw = info.num_cores * info.num_subcores
  idx3 = flat_idx.reshape(nw, n_chunks // nw, _CHUNK)
  out2 = _make_gather(dim_size, d, n_out, n_chunks, nw)(
      data.astype(jnp.float32), idx3)
  out = (
      out2.reshape(n_j, d // 8, n_bblk, 8, _CHUNK)
      .transpose(2, 4, 0, 1, 3)
      .reshape(n_out, n_j, d)
  )
  return out.astype(data.dtype)
